# Initial kernel scaffold; baseline (speedup 1.0000x reference)
import jax
import jax.numpy as jnp
from jax.experimental import pallas as pl


def kernel(nodes, edges, W_msg, W_edge, W_u, W_m, W_gate, W_gate_in, W_out):
    raise NotImplementedError("write your pallas kernel here")


# speedup vs baseline: 18.6772x; 18.6772x over previous
import functools

import jax
import jax.numpy as jnp
from jax.experimental import pallas as pl
from jax.experimental.pallas import tpu as pltpu

B, N, F = 32, 27, 128
HIDDEN = 128
D_EDGE = 16
MSG = 128
PASSES = 3
OUT = 128
NP = 32


def _mpnn_kernel(edges_t_ref, nodes_ref, W_msg_ref, W_edge_ref, W_u_ref,
                 W_m_ref, W_gate_ref, W_gate_in_ref, W_out_ref, out_ref):
    f32 = jnp.float32
    e = edges_t_ref[0]
    nodes = nodes_ref[0]

    adj_t = jnp.sum(e, axis=2)
    emask_t = (adj_t != 0.0).astype(f32)
    node_sum = jax.lax.dot_general(
        adj_t, jnp.ones((NP, 1), f32),
        (((0,), (0,)), ((), ())), preferred_element_type=f32)
    nmask = (node_sum != 0.0).astype(f32)

    e2 = e.reshape(NP * NP, D_EDGE)
    E2 = jnp.dot(e2, W_edge_ref[...], preferred_element_type=f32)
    E_t = E2.reshape(NP, NP, MSG)

    h = nodes
    for _ in range(PASSES):
        Hm = jnp.dot(h, W_msg_ref[...], preferred_element_type=f32)
        T = jnp.tanh(E_t + Hm[:, None, :]) * emask_t[:, :, None]
        msg = jnp.sum(T, axis=0) * nmask
        upd = jnp.tanh(
            jnp.dot(h, W_u_ref[...], preferred_element_type=f32)
            + jnp.dot(msg, W_m_ref[...], preferred_element_type=f32))
        h = jnp.where(nmask != 0.0, upd, h)

    gate = jax.nn.sigmoid(
        jnp.dot(h, W_gate_ref[...], preferred_element_type=f32)
        + jnp.dot(nodes, W_gate_in_ref[...], preferred_element_type=f32))
    emb = gate * jnp.dot(h, W_out_ref[...], preferred_element_type=f32)
    out_ref[0] = jnp.sum(emb * nmask, axis=0, keepdims=True)


@jax.jit
def kernel(nodes, edges, W_msg, W_edge, W_u, W_m, W_gate, W_gate_in, W_out):
    pad_n = NP - N
    nodes_p = jnp.pad(nodes, ((0, 0), (0, pad_n), (0, 0)))
    edges_t = jnp.pad(edges, ((0, 0), (0, pad_n), (0, pad_n), (0, 0)))
    edges_t = edges_t.transpose(0, 2, 1, 3)

    wspec = lambda *shape: pl.BlockSpec(shape, lambda b: (0,) * len(shape))
    out = pl.pallas_call(
        _mpnn_kernel,
        grid=(B,),
        in_specs=[
            pl.BlockSpec((1, NP, NP, D_EDGE), lambda b: (b, 0, 0, 0)),
            pl.BlockSpec((1, NP, F), lambda b: (b, 0, 0)),
            wspec(HIDDEN, MSG),
            wspec(D_EDGE, MSG),
            wspec(HIDDEN, HIDDEN),
            wspec(MSG, HIDDEN),
            wspec(HIDDEN, OUT),
            wspec(F, OUT),
            wspec(HIDDEN, OUT),
        ],
        out_specs=pl.BlockSpec((1, 1, OUT), lambda b: (b, 0, 0)),
        out_shape=jax.ShapeDtypeStruct((B, 1, OUT), jnp.float32),
        compiler_params=pltpu.CompilerParams(
            dimension_semantics=("arbitrary",)),
    )(edges_t, nodes_p, W_msg, W_edge, W_u, W_m, W_gate, W_gate_in, W_out)
    return out.reshape(B, OUT)


# speedup vs baseline: 29.0093x; 1.5532x over previous
import jax
import jax.numpy as jnp
from jax.experimental import pallas as pl
from jax.experimental.pallas import tpu as pltpu

B, N, F = 32, 27, 128
HIDDEN = 128
D_EDGE = 16
MSG = 128
PASSES = 3
OUT = 128
NP = 32
BB = 4


def _mpnn_kernel(edges_t_ref, nodes_ref, W_msg_ref, W_edge_ref, W_u_ref,
                 W_m_ref, W_gate_ref, W_gate_in_ref, W_out_ref, out_ref):
    f32 = jnp.float32
    e4 = edges_t_ref[...]
    nodes2 = nodes_ref[...].reshape(BB * NP, F)

    e2 = e4.reshape(BB * NP * NP, D_EDGE)
    adj = jnp.dot(e2, jnp.ones((D_EDGE, 1), f32), preferred_element_type=f32)
    emask4 = (adj != 0.0).astype(f32).reshape(BB, NP, NP, 1)
    node_sum = jnp.sum(adj.reshape(BB, NP, NP, 1), axis=1)
    nmask2 = (node_sum != 0.0).astype(f32).reshape(BB * NP, 1)

    E4 = jnp.dot(e2, W_edge_ref[...],
                 preferred_element_type=f32).reshape(BB, NP, NP, MSG)

    h2 = nodes2
    for _ in range(PASSES):
        Hm4 = jnp.dot(h2, W_msg_ref[...],
                      preferred_element_type=f32).reshape(BB, NP, 1, MSG)
        T = jnp.tanh(E4 + Hm4) * emask4
        msg2 = jnp.sum(T, axis=1).reshape(BB * NP, MSG)
        upd = jnp.tanh(
            jnp.dot(h2, W_u_ref[...], preferred_element_type=f32)
            + jnp.dot(msg2, W_m_ref[...], preferred_element_type=f32))
        h2 = jnp.where(nmask2 != 0.0, upd, h2)

    gate = jax.nn.sigmoid(
        jnp.dot(h2, W_gate_ref[...], preferred_element_type=f32)
        + jnp.dot(nodes2, W_gate_in_ref[...], preferred_element_type=f32))
    emb = gate * jnp.dot(h2, W_out_ref[...], preferred_element_type=f32)
    emb = (emb * nmask2).reshape(BB, NP, OUT)
    out_ref[...] = jnp.sum(emb, axis=1, keepdims=True)


@jax.jit
def kernel(nodes, edges, W_msg, W_edge, W_u, W_m, W_gate, W_gate_in, W_out):
    pad_n = NP - N
    nodes_p = jnp.pad(nodes, ((0, 0), (0, pad_n), (0, 0)))
    edges_t = jnp.pad(edges, ((0, 0), (0, pad_n), (0, pad_n), (0, 0)))
    edges_t = edges_t.transpose(0, 2, 1, 3)

    wspec = lambda *shape: pl.BlockSpec(shape, lambda b: (0,) * len(shape))
    out = pl.pallas_call(
        _mpnn_kernel,
        grid=(B // BB,),
        in_specs=[
            pl.BlockSpec((BB, NP, NP, D_EDGE), lambda b: (b, 0, 0, 0)),
            pl.BlockSpec((BB, NP, F), lambda b: (b, 0, 0)),
            wspec(HIDDEN, MSG),
            wspec(D_EDGE, MSG),
            wspec(HIDDEN, HIDDEN),
            wspec(MSG, HIDDEN),
            wspec(HIDDEN, OUT),
            wspec(F, OUT),
            wspec(HIDDEN, OUT),
        ],
        out_specs=pl.BlockSpec((BB, 1, OUT), lambda b: (b, 0, 0)),
        out_shape=jax.ShapeDtypeStruct((B, 1, OUT), jnp.float32),
        compiler_params=pltpu.CompilerParams(
            dimension_semantics=("arbitrary",)),
    )(edges_t, nodes_p, W_msg, W_edge, W_u, W_m, W_gate, W_gate_in, W_out)
    return out.reshape(B, OUT)
